# baseline (speedup 1.0000x reference)
import functools
import math

import jax
import jax.numpy as jnp
from jax import lax
from jax.experimental import pallas as pl
from jax.experimental.pallas import tpu as pltpu
from jax.experimental.pallas import tpu_sc as plsc

NUM_USERS = 50000
NUM_ITEMS = 50000
N_LAYERS = 3

_NC = 2
_NS = 16
_L = 16
_SUB = 128
_ZB = 1024


def _rup(x, m):
    return (x + m - 1) // m * m


@functools.lru_cache(maxsize=None)
def _make_segsum_rows(n_in, n_out, E, D, weighted):
    C = D // _L
    assert C % 2 == 0, "feature chunks must split evenly across 2 SCs"
    G = 8
    BATCH = G * _SUB
    NIP = _rup(n_in + 1, 16)
    NOP = _rup(n_out + 1, 16 * _NS)
    EP = _rup(E, 2 * _NS * BATCH)
    ept = EP // _NS
    nsub = ept // BATCH
    rpt = NOP // _NS
    kmax = C // _NC

    mesh = plsc.VectorSubcoreMesh(core_axis_name="c", subcore_axis_name="s")

    ESUB = EP // _SUB

    def body(xr_hbm, srcm_hbm, dstm_hbm, w_hbm, out_hbm,
             acc, sidx, didx, rows, wbuf, semi, semg, sems):
        c = lax.axis_index("c")
        s = lax.axis_index("s")
        e_base = s * ept
        r0 = s * rpt

        for k in range(kmax):
            cc = k * _NC + c

            def start_idx(b, p):
                row = (e_base + b * BATCH) // _SUB
                pltpu.async_copy(srcm_hbm.at[pl.ds(cc * ESUB + row, G)],
                                 sidx.at[p], semi.at[p])
                pltpu.async_copy(dstm_hbm.at[pl.ds(row, G)], didx.at[p],
                                 semi.at[p])
                if weighted:
                    pltpu.async_copy(
                        w_hbm.at[pl.ds(e_base + b * BATCH, BATCH)],
                        wbuf.at[p], semi.at[p])

            def wait_idx(p):
                pltpu.make_async_copy(srcm_hbm.at[pl.ds(0, G)],
                                      sidx.at[p], semi.at[p]).wait()
                pltpu.make_async_copy(dstm_hbm.at[pl.ds(0, G)],
                                      didx.at[p], semi.at[p]).wait()
                if weighted:
                    pltpu.make_async_copy(w_hbm.at[pl.ds(0, BATCH)],
                                          wbuf.at[p], semi.at[p]).wait()

            def zinit(i, _):
                rows[i] = jnp.zeros((_L,), jnp.float32)
                return 0
            lax.fori_loop(0, BATCH, zinit, 0)
            nz = rpt // BATCH
            for zi in range(nz):
                pltpu.sync_copy(rows, acc.at[pl.ds(r0 + zi * BATCH, BATCH)])
            rem = rpt - nz * BATCH
            if rem:
                pltpu.sync_copy(rows.at[pl.ds(0, rem)],
                                acc.at[pl.ds(r0 + nz * BATCH, rem)])
            plsc.subcore_barrier()

            start_idx(0, 0)

            def substep(b, p):
                @pl.when(b + 1 < nsub)
                def _():
                    start_idx(b + 1, 1 - p)
                wait_idx(p)
                gd = [pltpu.async_copy(xr_hbm.at[sidx.at[p, j]],
                                       rows.at[pl.ds(j * _SUB, _SUB)],
                                       semg)
                      for j in range(G)]
                for d in gd:
                    d.wait()
                if weighted:
                    def scale(j, _):
                        wsplat = plsc.load_gather(
                            wbuf.at[p], [jnp.broadcast_to(j, (_L,))])
                        rows[j] = rows[j] * wsplat
                        return 0
                    lax.fori_loop(0, BATCH, scale, 0)
                sd = [pltpu.async_copy(rows.at[pl.ds(j * _SUB, _SUB)],
                                       acc.at[didx.at[p, j]],
                                       sems, add=True)
                      for j in range(G)]
                for d in sd:
                    d.wait()

            def step(bb, _):
                substep(bb * 2, 0)
                substep(bb * 2 + 1, 1)
                return 0
            lax.fori_loop(0, nsub // 2, step, 0)
            plsc.subcore_barrier()

            pltpu.sync_copy(acc.at[pl.ds(r0, rpt)],
                            out_hbm.at[pl.ds(cc * NOP + r0, rpt)])
            plsc.subcore_barrier()

    scratch = [
        pltpu.VMEM_SHARED((NOP, _L), jnp.float32),
        pltpu.VMEM((2, G, _SUB), jnp.int32),
        pltpu.VMEM((2, G, _SUB), jnp.int32),
        pltpu.VMEM((BATCH, _L), jnp.float32),
        pltpu.VMEM((2, BATCH) if weighted else (2, 8), jnp.float32),
        pltpu.SemaphoreType.DMA((2,)),
        pltpu.SemaphoreType.DMA,
        pltpu.SemaphoreType.DMA,
    ]

    kern = pl.kernel(
        body,
        out_type=jax.ShapeDtypeStruct((C * NOP, _L), jnp.float32),
        mesh=mesh,
        scratch_types=scratch,
        compiler_params=pltpu.CompilerParams(
            use_tc_tiling_on_sc=False, needs_layout_passes=False),
        name=f"sc_segsum_rows_{n_in}_{n_out}_{E}_{D}_{weighted}",
    )

    def run(Xc, src, dst, w=None):
        Xr = Xc.reshape(C * n_in, _L)
        srcp = jnp.pad(src, (0, EP - E))
        srcm = (jnp.arange(C, dtype=jnp.int32)[:, None] * n_in
                + srcp[None, :]).reshape(C * EP // _SUB, _SUB)
        dstp = jnp.pad(dst, (0, EP - E),
                       constant_values=NOP - 1).reshape(EP // _SUB, _SUB)
        if weighted:
            wp = jnp.pad(w, (0, EP - E))
        else:
            wp = jnp.zeros((EP,), jnp.float32)
        out = kern(Xr, srcm, dstp, wp)
        return out.reshape(C, NOP, _L)[:, :n_out]

    return run


def _segsum_rows(Xc, src, dst, n_out, w=None):
    C, n_in, _ = Xc.shape
    E = src.shape[0]
    fn = _make_segsum_rows(n_in, n_out, E, C * _L, w is not None)
    return fn(Xc, src, dst, w)


def _to_chunks(X):
    n, D = X.shape
    return X.reshape(n, D // _L, _L).transpose(1, 0, 2)


def _from_chunks(Xc):
    C, n, _ = Xc.shape
    return Xc.transpose(1, 0, 2).reshape(n, C * _L)


@functools.lru_cache(maxsize=None)
def _make_segsum_scalar(n_out, E, gather, n_tbl):
    TR = _rup((n_out + 1 + 15) // 16, 128)
    TRG = (n_tbl + 15) // 16 if gather else 1
    G = 8
    BATCH = G * _SUB
    EP = _rup(E, 2 * _NC * _NS * BATCH)
    ept = EP // (_NC * _NS)
    nsub = ept // BATCH
    rpt = TR // _NS
    nblk = TR // 128

    mesh = plsc.VectorSubcoreMesh(core_axis_name="c", subcore_axis_name="s")

    def body(didx_hbm, gidx_hbm, qtbl_hbm, out_hbm,
             acc, tbl, qtbl, dbuf, gbuf, iota, semi, sem):
        c = lax.axis_index("c")
        s = lax.axis_index("s")
        wid = c * _NS + s
        r0 = s * rpt

        if gather:
            pltpu.sync_copy(qtbl_hbm, qtbl)

        def zrow(i, _):
            tbl[i] = jnp.zeros((_L,), jnp.float32)
            return 0
        lax.fori_loop(0, TR, zrow, 0)
        pltpu.sync_copy(tbl.at[pl.ds(0, rpt)], acc.at[pl.ds(r0, rpt)])
        plsc.subcore_barrier()

        def start_idx(b, p):
            row = (wid * ept + b * BATCH) // _SUB
            pltpu.async_copy(didx_hbm.at[pl.ds(row, G)], dbuf.at[p],
                             semi.at[p])
            if gather:
                pltpu.async_copy(gidx_hbm.at[pl.ds(row, G)], gbuf.at[p],
                                 semi.at[p])

        def wait_idx(p):
            pltpu.make_async_copy(didx_hbm.at[pl.ds(0, G)], dbuf.at[p],
                                  semi.at[p]).wait()
            if gather:
                pltpu.make_async_copy(gidx_hbm.at[pl.ds(0, G)], gbuf.at[p],
                                      semi.at[p]).wait()

        start_idx(0, 0)

        def substep(b, p):
            @pl.when(b + 1 < nsub)
            def _():
                start_idx(b + 1, 1 - p)
            wait_idx(p)
            for j in range(G):
                for g in range(_SUB // _L):
                    iv = dbuf[p, j, pl.ds(g * _L, _L)]
                    ir = lax.shift_right_logical(iv, 4)
                    ic = jnp.bitwise_and(iv, 15)
                    if gather:
                        gv = gbuf[p, j, pl.ds(g * _L, _L)]
                        val = plsc.load_gather(
                            qtbl, [lax.shift_right_logical(gv, 4),
                                   jnp.bitwise_and(gv, 15)])
                    else:
                        val = jnp.ones((_L,), jnp.float32)
                    plsc.addupdate_scatter(tbl, [ir, ic], val)

        def step(bb, _):
            substep(bb * 2, 0)
            substep(bb * 2 + 1, 1)
            return 0
        lax.fori_loop(0, nsub // 2, step, 0)

        def blk(m, _):
            for kk in range(128 // _L):
                iota[pl.ds(kk * _L, _L)] = (
                    m * 128 + kk * _L
                    + lax.broadcasted_iota(jnp.int32, (_L,), 0))
            pltpu.sync_copy(tbl.at[pl.ds(m * 128, 128)], acc.at[iota],
                            add=True)
            return 0
        lax.fori_loop(0, nblk, blk, 0)
        plsc.subcore_barrier()

        pltpu.sync_copy(acc.at[pl.ds(r0, rpt)],
                        out_hbm.at[pl.ds(c * TR + r0, rpt)])

    scratch = [
        pltpu.VMEM_SHARED((TR, _L), jnp.float32),
        pltpu.VMEM((TR, _L), jnp.float32),
        pltpu.VMEM((TRG, _L), jnp.float32),
        pltpu.VMEM((2, G, _SUB), jnp.int32),
        pltpu.VMEM((2, G, _SUB) if gather else (2, 1, 8), jnp.int32),
        pltpu.VMEM((128,), jnp.int32),
        pltpu.SemaphoreType.DMA((2,)),
        pltpu.SemaphoreType.DMA,
    ]

    kern = pl.kernel(
        body,
        out_type=jax.ShapeDtypeStruct((_NC * TR, _L), jnp.float32),
        mesh=mesh,
        scratch_types=scratch,
        compiler_params=pltpu.CompilerParams(
            use_tc_tiling_on_sc=False, needs_layout_passes=False),
        name=f"sc_segsum_scalar_{n_out}_{E}_{gather}",
    )

    def run(idx, gidx=None, q=None):
        idxp = jnp.pad(idx, (0, EP - E),
                       constant_values=TR * _L - 1).reshape(EP // _SUB, _SUB)
        if gather:
            gidxp = jnp.pad(gidx, (0, EP - E)).reshape(EP // _SUB, _SUB)
            qp = jnp.pad(q, (0, TRG * _L - n_tbl)).reshape(TRG, _L)
        else:
            gidxp = jnp.zeros((EP // _SUB, _SUB), jnp.int32)
            qp = jnp.zeros((TRG, _L), jnp.float32)
        out = kern(idxp, gidxp, qp)
        out = out.reshape(_NC, TR * _L)
        return (out[0] + out[1])[:n_out]

    return run


def _segsum_scalar(idx, n_out, gidx=None, q=None):
    gather = q is not None
    n_tbl = q.shape[0] if gather else 0
    fn = _make_segsum_scalar(n_out, idx.shape[0], gather, n_tbl)
    return fn(idx, gidx, q)


@functools.lru_cache(maxsize=None)
def _make_matmul(N, K, F, act, want_max=False, in_chunked=False,
                 out_chunked=False, BN=512):
    NP = _rup(N, BN)
    CK = K // _L
    CF = F // _L

    def body(x_ref, w_ref, b_ref, *orefs):
        if in_chunked:
            x = jnp.concatenate([x_ref[k] for k in range(CK)], axis=-1)
        else:
            x = x_ref[...]
        o = jnp.dot(x, w_ref[...],
                    preferred_element_type=jnp.float32) + b_ref[0:1, :]
        if act == "relu":
            o = jnp.maximum(o, 0.0)
        elif act == "leaky":
            o = jnp.where(o > 0, o, 0.01 * o)
        elif act == "elu":
            o = jnp.where(o > 0, o, jnp.exp(jnp.minimum(o, 0.0)) - 1.0)
        if out_chunked:
            orefs[0][...] = jnp.stack(
                [o[:, k * _L:(k + 1) * _L] for k in range(CF)])
        else:
            orefs[0][...] = o
        if want_max:
            i = pl.program_id(0)

            @pl.when(i == 0)
            def _():
                orefs[1][...] = jnp.full((8, F), -1e30, jnp.float32)
            m = jnp.max(o, axis=0, keepdims=True)
            orefs[1][...] = jnp.maximum(orefs[1][...],
                                        jnp.broadcast_to(m, (8, F)))

    if out_chunked:
        out_shapes = [jax.ShapeDtypeStruct((CF, NP, _L), jnp.float32)]
        out_specs = [pl.BlockSpec((CF, BN, _L), lambda i: (0, i, 0))]
    else:
        out_shapes = [jax.ShapeDtypeStruct((NP, F), jnp.float32)]
        out_specs = [pl.BlockSpec((BN, F), lambda i: (i, 0))]
    if want_max:
        out_shapes.append(jax.ShapeDtypeStruct((8, F), jnp.float32))
        out_specs.append(pl.BlockSpec((8, F), lambda i: (0, 0)))

    if in_chunked:
        x_spec = pl.BlockSpec((CK, BN, _L), lambda i: (0, i, 0))
    else:
        x_spec = pl.BlockSpec((BN, K), lambda i: (i, 0))

    call = pl.pallas_call(
        body,
        grid=(NP // BN,),
        in_specs=[
            x_spec,
            pl.BlockSpec((K, F), lambda i: (0, 0)),
            pl.BlockSpec((8, F), lambda i: (0, 0)),
        ],
        out_specs=out_specs,
        out_shape=out_shapes,
    )

    def run(x, W, b):
        if in_chunked:
            xp = jnp.pad(x, ((0, 0), (0, NP - N), (0, 0)))
        else:
            xp = jnp.pad(x, ((0, NP - N), (0, 0)))
        bt = jnp.broadcast_to(b[None, :], (8, F))
        res = call(xp, W, bt)
        out = res[0][:, :N] if out_chunked else res[0][:N]
        if want_max:
            return out, res[1]
        return out

    return run


def _matmul(x, W, b, act="none", want_max=False, out_chunked=False):
    in_chunked = x.ndim == 3
    if in_chunked:
        N = x.shape[1]
        K = x.shape[0] * _L
    else:
        N, K = x.shape
    F = W.shape[1]
    return _make_matmul(N, K, F, act, want_max, in_chunked,
                        out_chunked)(x, W, b)


@functools.lru_cache(maxsize=None)
def _make_pairdot(N, D, BN=1024):
    def body(a_ref, b_ref, o_ref):
        o = jnp.sum(a_ref[...] * b_ref[...], axis=1, keepdims=True)
        o_ref[...] = jnp.broadcast_to(o, (BN, 128))

    call = pl.pallas_call(
        body,
        grid=(N // BN,),
        in_specs=[
            pl.BlockSpec((BN, D), lambda i: (i, 0)),
            pl.BlockSpec((BN, D), lambda i: (i, 0)),
        ],
        out_specs=pl.BlockSpec((BN, 128), lambda i: (i, 0)),
        out_shape=jax.ShapeDtypeStruct((N, 128), jnp.float32),
    )

    def run(a, b):
        return call(a, b)[:, 0]

    return run


def kernel(u_emb, i_emb, hyper_edge_emb, W_edge, b_edge, W_conv, b_conv,
           W_or, b_or, a_or, W_ee, b_ee, a_ee, W1, b1, W2, b2, W3, b3,
           W_theta, b_theta, pagerank_weight, ui_edge_index, social_incidence,
           or_incidence, ee_incidence, or_x, ee_x):
    NU = NUM_USERS
    n_all = NU + NUM_ITEMS

    u_idx = ui_edge_index[0]
    i_idx = ui_edge_index[1] + NU
    src = jnp.concatenate([u_idx, i_idx])
    dst = jnp.concatenate([i_idx, u_idx])
    deg = _segsum_scalar(dst, n_all)
    dinv = jnp.where(deg > 0, 1.0 / jnp.sqrt(jnp.maximum(deg, 1.0)), 0.0)
    dv3 = dinv[None, :, None]
    X = jnp.concatenate([u_emb, i_emb], axis=0)
    Xc = _to_chunks(X)
    acc = Xc
    Xl = Xc
    for _ in range(N_LAYERS):
        Xl = _segsum_rows(Xl * dv3, src, dst, n_all) * dv3
        acc = acc + Xl
    u_embs = (acc / (N_LAYERS + 1.0))[:, :NU]

    soc_v, soc_e = social_incidence[0], social_incidence[1]
    dv = _segsum_scalar(soc_v, NU)
    de = _segsum_scalar(soc_e, NU)
    dvis = jnp.where(dv > 0, 1.0 / jnp.sqrt(jnp.maximum(dv, 1.0)), 0.0)
    dei = jnp.where(de > 0, 1.0 / jnp.maximum(de, 1.0), 0.0)
    Hc = jnp.stack([hyper_edge_emb * dvis[:, None],
                    jnp.zeros_like(hyper_edge_emb)])
    Ye = _segsum_rows(Hc, soc_v, soc_e, NU) * dei[None, :, None]
    edge_x = (_segsum_rows(Ye, soc_e, soc_v, NU) * dvis[None, :, None])[0]
    u = u_embs + _matmul(edge_x, W_edge, b_edge, out_chunked=True)

    dem = jnp.maximum(de, 1.0)
    Ye2 = _segsum_rows(u, soc_v, soc_e, NU) / dem[None, :, None]
    Magg = _segsum_rows(Ye2, soc_e, soc_v, NU)
    hyper = 3.0 * _matmul(u + Magg, W_conv, b_conv, act="relu",
                          out_chunked=True)

    def unigat(v_idx, e_idx, W, b, a):
        Xt = _matmul(u, W, b, out_chunked=True)
        dee = jnp.maximum(_segsum_scalar(e_idx, NU), 1.0)
        Ye3 = _segsum_rows(Xt, v_idx, e_idx, NU) / dee[None, :, None]
        ap = jnp.pad(a[:, None], ((0, 0), (0, 127)))
        alpha_full, amax = _matmul(Ye3, ap, jnp.zeros((128,), jnp.float32),
                                   want_max=True)
        alpha = alpha_full[:, 0]
        am = jnp.max(amax)
        sm = jnp.where(am > 0, am, 0.2 * am)
        q = jnp.exp(jnp.where(alpha > 0, alpha, 0.2 * alpha) - sm)
        denom = _segsum_scalar(v_idx, NU, gidx=e_idx, q=q)
        num = _segsum_rows(q[None, :, None] * Ye3, e_idx, v_idx, NU)
        out = num / jnp.maximum(denom, 1e-12)[None, :, None]
        return jnp.where(out > 0, out, jnp.exp(jnp.minimum(out, 0.0)) - 1.0)

    or_gat = unigat(or_incidence[0], or_incidence[1], W_or, b_or, a_or)
    ee_gat = unigat(ee_incidence[0], ee_incidence[1], W_ee, b_ee, a_ee)

    def mlp(xc):
        x = _matmul(xc, W1, b1, act="leaky")
        x = _matmul(x, W2, b2, act="leaky")
        return _matmul(x, W3, b3, out_chunked=True)

    trustor_all = mlp(jnp.concatenate([or_gat, hyper], axis=0))
    trustee_all = mlp(jnp.concatenate([ee_gat, hyper], axis=0))
    hyper_x = _matmul(hyper, W_theta, b_theta, act="relu", out_chunked=True)
    B = or_x.shape[0]
    pair_iota = jnp.arange(B, dtype=jnp.int32)
    trustor = _from_chunks(_segsum_rows(trustor_all + hyper_x,
                                        or_x, pair_iota, B))
    trustee = _from_chunks(_segsum_rows(trustee_all + hyper_x,
                                        ee_x, pair_iota, B))
    output = _make_pairdot(B, trustor.shape[1])(trustor, trustee)
    return trustor, trustee, output


# speedup vs baseline: 1.0463x; 1.0463x over previous
import functools
import math

import jax
import jax.numpy as jnp
from jax import lax
from jax.experimental import pallas as pl
from jax.experimental.pallas import tpu as pltpu
from jax.experimental.pallas import tpu_sc as plsc

NUM_USERS = 50000
NUM_ITEMS = 50000
N_LAYERS = 3

_NC = 2
_NS = 16
_L = 16
_SUB = 128
_ZB = 1024


def _rup(x, m):
    return (x + m - 1) // m * m


@functools.lru_cache(maxsize=None)
def _make_segsum_rows(n_in, n_out, E, D, weighted):
    C = D // _L
    assert C % 2 == 0, "feature chunks must split evenly across 2 SCs"
    G = 8
    BATCH = G * _SUB
    NIP = _rup(n_in + 1, 16)
    NOP = _rup(n_out + 1, 16 * _NS)
    EP = _rup(E, 2 * _NS * BATCH)
    ept = EP // _NS
    nsub = ept // BATCH
    rpt = NOP // _NS
    kmax = C // _NC

    mesh = plsc.VectorSubcoreMesh(core_axis_name="c", subcore_axis_name="s")

    ESUB = EP // _SUB

    def body(xr_hbm, srcm_hbm, dstm_hbm, w_hbm, out_hbm,
             acc, sidx, didx, rows, wbuf, semi, semg, sems):
        c = lax.axis_index("c")
        s = lax.axis_index("s")
        e_base = s * ept
        r0 = s * rpt

        for k in range(kmax):
            cc = k * _NC + c

            def start_idx(b, p):
                row = (e_base + b * BATCH) // _SUB
                pltpu.async_copy(srcm_hbm.at[pl.ds(cc * ESUB + row, G)],
                                 sidx.at[p], semi.at[p])
                pltpu.async_copy(dstm_hbm.at[pl.ds(row, G)], didx.at[p],
                                 semi.at[p])
                if weighted:
                    pltpu.async_copy(
                        w_hbm.at[pl.ds(e_base + b * BATCH, BATCH)],
                        wbuf.at[p], semi.at[p])

            def wait_idx(p):
                pltpu.make_async_copy(srcm_hbm.at[pl.ds(0, G)],
                                      sidx.at[p], semi.at[p]).wait()
                pltpu.make_async_copy(dstm_hbm.at[pl.ds(0, G)],
                                      didx.at[p], semi.at[p]).wait()
                if weighted:
                    pltpu.make_async_copy(w_hbm.at[pl.ds(0, BATCH)],
                                          wbuf.at[p], semi.at[p]).wait()

            def zinit(i, _):
                rows[i] = jnp.zeros((_L,), jnp.float32)
                return 0
            lax.fori_loop(0, BATCH, zinit, 0)
            nz = rpt // BATCH
            for zi in range(nz):
                pltpu.sync_copy(rows, acc.at[pl.ds(r0 + zi * BATCH, BATCH)])
            rem = rpt - nz * BATCH
            if rem:
                pltpu.sync_copy(rows.at[pl.ds(0, rem)],
                                acc.at[pl.ds(r0 + nz * BATCH, rem)])
            plsc.subcore_barrier()

            start_idx(0, 0)

            def substep(b, p):
                @pl.when(b + 1 < nsub)
                def _():
                    start_idx(b + 1, 1 - p)
                wait_idx(p)
                gd = [pltpu.async_copy(xr_hbm.at[sidx.at[p, j]],
                                       rows.at[pl.ds(j * _SUB, _SUB)],
                                       semg)
                      for j in range(G)]
                for d in gd:
                    d.wait()
                if weighted:
                    def scale(j, _):
                        wsplat = plsc.load_gather(
                            wbuf.at[p], [jnp.broadcast_to(j, (_L,))])
                        rows[j] = rows[j] * wsplat
                        return 0
                    lax.fori_loop(0, BATCH, scale, 0)
                sd = [pltpu.async_copy(rows.at[pl.ds(j * _SUB, _SUB)],
                                       acc.at[didx.at[p, j]],
                                       sems, add=True)
                      for j in range(G)]
                for d in sd:
                    d.wait()

            def step(bb, _):
                substep(bb * 2, 0)
                substep(bb * 2 + 1, 1)
                return 0
            lax.fori_loop(0, nsub // 2, step, 0)
            plsc.subcore_barrier()

            pltpu.sync_copy(acc.at[pl.ds(r0, rpt)],
                            out_hbm.at[pl.ds(cc * NOP + r0, rpt)])
            plsc.subcore_barrier()

    scratch = [
        pltpu.VMEM_SHARED((NOP, _L), jnp.float32),
        pltpu.VMEM((2, G, _SUB), jnp.int32),
        pltpu.VMEM((2, G, _SUB), jnp.int32),
        pltpu.VMEM((BATCH, _L), jnp.float32),
        pltpu.VMEM((2, BATCH) if weighted else (2, 8), jnp.float32),
        pltpu.SemaphoreType.DMA((2,)),
        pltpu.SemaphoreType.DMA,
        pltpu.SemaphoreType.DMA,
    ]

    kern = pl.kernel(
        body,
        out_type=jax.ShapeDtypeStruct((C * NOP, _L), jnp.float32),
        mesh=mesh,
        scratch_types=scratch,
        compiler_params=pltpu.CompilerParams(
            use_tc_tiling_on_sc=False, needs_layout_passes=False),
        name=f"sc_segsum_rows_{n_in}_{n_out}_{E}_{D}_{weighted}",
    )

    def run(X, src, dst, w=None):
        Xr = X.reshape(n_in, C, _L).transpose(1, 0, 2).reshape(C * n_in, _L)
        srcp = jnp.pad(src, (0, EP - E))
        srcm = (jnp.arange(C, dtype=jnp.int32)[:, None] * n_in
                + srcp[None, :]).reshape(C * EP // _SUB, _SUB)
        dstp = jnp.pad(dst, (0, EP - E),
                       constant_values=NOP - 1).reshape(EP // _SUB, _SUB)
        if weighted:
            wp = jnp.pad(w, (0, EP - E))
        else:
            wp = jnp.zeros((EP,), jnp.float32)
        out = kern(Xr, srcm, dstp, wp)
        out = out.reshape(C, NOP, _L)[:, :n_out].transpose(1, 0, 2)
        return out.reshape(n_out, D)

    return run


def _segsum_rows(X, src, dst, n_out, w=None):
    n_in, D = X.shape
    E = src.shape[0]
    fn = _make_segsum_rows(n_in, n_out, E, D, w is not None)
    return fn(X, src, dst, w)


@functools.lru_cache(maxsize=None)
def _make_segsum_scalar(n_out, E, gather, n_tbl):
    TR = _rup((n_out + 1 + 15) // 16, 128)
    TRG = (n_tbl + 15) // 16 if gather else 1
    G = 8
    BATCH = G * _SUB
    EP = _rup(E, 2 * _NC * _NS * BATCH)
    ept = EP // (_NC * _NS)
    nsub = ept // BATCH
    rpt = TR // _NS
    nblk = TR // 128

    mesh = plsc.VectorSubcoreMesh(core_axis_name="c", subcore_axis_name="s")

    def body(didx_hbm, gidx_hbm, qtbl_hbm, out_hbm,
             acc, tbl, qtbl, dbuf, gbuf, iota, semi, sem):
        c = lax.axis_index("c")
        s = lax.axis_index("s")
        wid = c * _NS + s
        r0 = s * rpt

        if gather:
            pltpu.sync_copy(qtbl_hbm, qtbl)

        def zrow(i, _):
            tbl[i] = jnp.zeros((_L,), jnp.float32)
            return 0
        lax.fori_loop(0, TR, zrow, 0)
        pltpu.sync_copy(tbl.at[pl.ds(0, rpt)], acc.at[pl.ds(r0, rpt)])
        plsc.subcore_barrier()

        def start_idx(b, p):
            row = (wid * ept + b * BATCH) // _SUB
            pltpu.async_copy(didx_hbm.at[pl.ds(row, G)], dbuf.at[p],
                             semi.at[p])
            if gather:
                pltpu.async_copy(gidx_hbm.at[pl.ds(row, G)], gbuf.at[p],
                                 semi.at[p])

        def wait_idx(p):
            pltpu.make_async_copy(didx_hbm.at[pl.ds(0, G)], dbuf.at[p],
                                  semi.at[p]).wait()
            if gather:
                pltpu.make_async_copy(gidx_hbm.at[pl.ds(0, G)], gbuf.at[p],
                                      semi.at[p]).wait()

        start_idx(0, 0)

        def substep(b, p):
            @pl.when(b + 1 < nsub)
            def _():
                start_idx(b + 1, 1 - p)
            wait_idx(p)
            for j in range(G):
                for g in range(_SUB // _L):
                    iv = dbuf[p, j, pl.ds(g * _L, _L)]
                    ir = lax.shift_right_logical(iv, 4)
                    ic = jnp.bitwise_and(iv, 15)
                    if gather:
                        gv = gbuf[p, j, pl.ds(g * _L, _L)]
                        val = plsc.load_gather(
                            qtbl, [lax.shift_right_logical(gv, 4),
                                   jnp.bitwise_and(gv, 15)])
                    else:
                        val = jnp.ones((_L,), jnp.float32)
                    plsc.addupdate_scatter(tbl, [ir, ic], val)

        def step(bb, _):
            substep(bb * 2, 0)
            substep(bb * 2 + 1, 1)
            return 0
        lax.fori_loop(0, nsub // 2, step, 0)

        def blk(m, _):
            for kk in range(128 // _L):
                iota[pl.ds(kk * _L, _L)] = (
                    m * 128 + kk * _L
                    + lax.broadcasted_iota(jnp.int32, (_L,), 0))
            pltpu.sync_copy(tbl.at[pl.ds(m * 128, 128)], acc.at[iota],
                            add=True)
            return 0
        lax.fori_loop(0, nblk, blk, 0)
        plsc.subcore_barrier()

        pltpu.sync_copy(acc.at[pl.ds(r0, rpt)],
                        out_hbm.at[pl.ds(c * TR + r0, rpt)])

    scratch = [
        pltpu.VMEM_SHARED((TR, _L), jnp.float32),
        pltpu.VMEM((TR, _L), jnp.float32),
        pltpu.VMEM((TRG, _L), jnp.float32),
        pltpu.VMEM((2, G, _SUB), jnp.int32),
        pltpu.VMEM((2, G, _SUB) if gather else (2, 1, 8), jnp.int32),
        pltpu.VMEM((128,), jnp.int32),
        pltpu.SemaphoreType.DMA((2,)),
        pltpu.SemaphoreType.DMA,
    ]

    kern = pl.kernel(
        body,
        out_type=jax.ShapeDtypeStruct((_NC * TR, _L), jnp.float32),
        mesh=mesh,
        scratch_types=scratch,
        compiler_params=pltpu.CompilerParams(
            use_tc_tiling_on_sc=False, needs_layout_passes=False),
        name=f"sc_segsum_scalar_{n_out}_{E}_{gather}",
    )

    def run(idx, gidx=None, q=None):
        idxp = jnp.pad(idx, (0, EP - E),
                       constant_values=TR * _L - 1).reshape(EP // _SUB, _SUB)
        if gather:
            gidxp = jnp.pad(gidx, (0, EP - E)).reshape(EP // _SUB, _SUB)
            qp = jnp.pad(q, (0, TRG * _L - n_tbl)).reshape(TRG, _L)
        else:
            gidxp = jnp.zeros((EP // _SUB, _SUB), jnp.int32)
            qp = jnp.zeros((TRG, _L), jnp.float32)
        out = kern(idxp, gidxp, qp)
        out = out.reshape(_NC, TR * _L)
        return (out[0] + out[1])[:n_out]

    return run


def _segsum_scalar(idx, n_out, gidx=None, q=None):
    gather = q is not None
    n_tbl = q.shape[0] if gather else 0
    fn = _make_segsum_scalar(n_out, idx.shape[0], gather, n_tbl)
    return fn(idx, gidx, q)


@functools.lru_cache(maxsize=None)
def _make_matmul(N, K, F, act, want_max=False, BN=512):
    NP = _rup(N, BN)

    def body(x_ref, w_ref, b_ref, *orefs):
        o = jnp.dot(x_ref[...], w_ref[...],
                    preferred_element_type=jnp.float32) + b_ref[0:1, :]
        if act == "relu":
            o = jnp.maximum(o, 0.0)
        elif act == "leaky":
            o = jnp.where(o > 0, o, 0.01 * o)
        elif act == "elu":
            o = jnp.where(o > 0, o, jnp.exp(jnp.minimum(o, 0.0)) - 1.0)
        orefs[0][...] = o
        if want_max:
            i = pl.program_id(0)

            @pl.when(i == 0)
            def _():
                orefs[1][...] = jnp.full((8, F), -1e30, jnp.float32)
            m = jnp.max(o, axis=0, keepdims=True)
            orefs[1][...] = jnp.maximum(orefs[1][...],
                                        jnp.broadcast_to(m, (8, F)))

    out_shapes = [jax.ShapeDtypeStruct((NP, F), jnp.float32)]
    out_specs = [pl.BlockSpec((BN, F), lambda i: (i, 0))]
    if want_max:
        out_shapes.append(jax.ShapeDtypeStruct((8, F), jnp.float32))
        out_specs.append(pl.BlockSpec((8, F), lambda i: (0, 0)))

    call = pl.pallas_call(
        body,
        grid=(NP // BN,),
        in_specs=[
            pl.BlockSpec((BN, K), lambda i: (i, 0)),
            pl.BlockSpec((K, F), lambda i: (0, 0)),
            pl.BlockSpec((8, F), lambda i: (0, 0)),
        ],
        out_specs=out_specs,
        out_shape=out_shapes,
    )

    def run(x, W, b):
        xp = jnp.pad(x, ((0, NP - N), (0, 0)))
        bt = jnp.broadcast_to(b[None, :], (8, F))
        res = call(xp, W, bt)
        if want_max:
            return res[0][:N], res[1]
        return res[0][:N]

    return run


def _matmul(x, W, b, act="none", want_max=False):
    N, K = x.shape
    F = W.shape[1]
    return _make_matmul(N, K, F, act, want_max)(x, W, b)


@functools.lru_cache(maxsize=None)
def _make_pairdot(N, D, BN=1024):
    def body(a_ref, b_ref, o_ref):
        o = jnp.sum(a_ref[...] * b_ref[...], axis=1, keepdims=True)
        o_ref[...] = jnp.broadcast_to(o, (BN, 128))

    call = pl.pallas_call(
        body,
        grid=(N // BN,),
        in_specs=[
            pl.BlockSpec((BN, D), lambda i: (i, 0)),
            pl.BlockSpec((BN, D), lambda i: (i, 0)),
        ],
        out_specs=pl.BlockSpec((BN, 128), lambda i: (i, 0)),
        out_shape=jax.ShapeDtypeStruct((N, 128), jnp.float32),
    )

    def run(a, b):
        return call(a, b)[:, 0]

    return run


def kernel(u_emb, i_emb, hyper_edge_emb, W_edge, b_edge, W_conv, b_conv,
           W_or, b_or, a_or, W_ee, b_ee, a_ee, W1, b1, W2, b2, W3, b3,
           W_theta, b_theta, pagerank_weight, ui_edge_index, social_incidence,
           or_incidence, ee_incidence, or_x, ee_x):
    NU = NUM_USERS
    n_all = NU + NUM_ITEMS

    u_idx = ui_edge_index[0]
    i_idx = ui_edge_index[1] + NU
    src = jnp.concatenate([u_idx, i_idx])
    dst = jnp.concatenate([i_idx, u_idx])
    deg = _segsum_scalar(dst, n_all)
    dinv = jnp.where(deg > 0, 1.0 / jnp.sqrt(jnp.maximum(deg, 1.0)), 0.0)
    X = jnp.concatenate([u_emb, i_emb], axis=0)
    acc = X
    Xl = X
    for _ in range(N_LAYERS):
        Xl = _segsum_rows(Xl * dinv[:, None], src, dst, n_all) * dinv[:, None]
        acc = acc + Xl
    u_embs = (acc / (N_LAYERS + 1.0))[:NU]

    soc_v, soc_e = social_incidence[0], social_incidence[1]
    dv = _segsum_scalar(soc_v, NU)
    de = _segsum_scalar(soc_e, NU)
    dvis = jnp.where(dv > 0, 1.0 / jnp.sqrt(jnp.maximum(dv, 1.0)), 0.0)
    dei = jnp.where(de > 0, 1.0 / jnp.maximum(de, 1.0), 0.0)
    Hp = jnp.pad(hyper_edge_emb, ((0, 0), (0, 16)))
    Ye = _segsum_rows(Hp * dvis[:, None], soc_v, soc_e, NU) * dei[:, None]
    edge_x = (_segsum_rows(Ye, soc_e, soc_v, NU) * dvis[:, None])[:, :16]
    u = u_embs + _matmul(edge_x, W_edge, b_edge)

    dem = jnp.maximum(de, 1.0)
    Ye2 = _segsum_rows(u, soc_v, soc_e, NU) / dem[:, None]
    Magg = _segsum_rows(Ye2, soc_e, soc_v, NU)
    hyper = 3.0 * _matmul(u + Magg, W_conv, b_conv, act="relu")

    def unigat(v_idx, e_idx, W, b, a):
        Xt = _matmul(u, W, b)
        dee = jnp.maximum(_segsum_scalar(e_idx, NU), 1.0)
        Ye3 = _segsum_rows(Xt, v_idx, e_idx, NU) / dee[:, None]
        ap = jnp.pad(a[:, None], ((0, 0), (0, 127)))
        alpha_full, amax = _matmul(Ye3, ap, jnp.zeros((128,), jnp.float32),
                                   want_max=True)
        alpha = alpha_full[:, 0]
        am = jnp.max(amax)
        sm = jnp.where(am > 0, am, 0.2 * am)
        q = jnp.exp(jnp.where(alpha > 0, alpha, 0.2 * alpha) - sm)
        denom = _segsum_scalar(v_idx, NU, gidx=e_idx, q=q)
        num = _segsum_rows(q[:, None] * Ye3, e_idx, v_idx, NU)
        out = num / jnp.maximum(denom, 1e-12)[:, None]
        return jnp.where(out > 0, out, jnp.exp(jnp.minimum(out, 0.0)) - 1.0)

    or_gat = unigat(or_incidence[0], or_incidence[1], W_or, b_or, a_or)
    ee_gat = unigat(ee_incidence[0], ee_incidence[1], W_ee, b_ee, a_ee)

    def mlp(x):
        x = _matmul(x, W1, b1, act="leaky")
        x = _matmul(x, W2, b2, act="leaky")
        return _matmul(x, W3, b3)

    trustor_all = mlp(jnp.concatenate([or_gat, hyper], axis=1))
    trustee_all = mlp(jnp.concatenate([ee_gat, hyper], axis=1))
    hyper_x = _matmul(hyper, W_theta, b_theta, act="relu")
    B = or_x.shape[0]
    pair_iota = jnp.arange(B, dtype=jnp.int32)
    trustor = _segsum_rows(trustor_all + hyper_x, or_x, pair_iota, B)
    trustee = _segsum_rows(trustee_all + hyper_x, ee_x, pair_iota, B)
    output = _make_pairdot(B, trustor.shape[1])(trustor, trustee)
    return trustor, trustee, output
